# Initial kernel scaffold; baseline (speedup 1.0000x reference)
import jax
import jax.numpy as jnp
from jax.experimental import pallas as pl


def kernel(boxes_xyxy, obj_logits, cls_logits, field):
    raise NotImplementedError("write your pallas kernel here")


# speedup vs baseline: 131.1695x; 131.1695x over previous
import functools

import numpy as np

import jax
import jax.numpy as jnp
from jax import lax
from jax.experimental import pallas as pl
from jax.experimental.pallas import tpu as pltpu
from jax.experimental.pallas import tpu_sc as plsc

B, A, C = 4, 8400, 80
H = W = 640
HF = WF = 160
POOL = 7
LAMBDA_OBJ = 0.5
LAMBDA_CLS = 0.25
LAMBDA_SMALL = 1.0
A_MIN = 0.005
SCORE_THR = 0.25
TOP_K = 300

AP = 8448
ROWS = AP // 128
KP = 304
CHUNK = 528
NEG_FILL = -1e9
PAD_FILL = -3e9


SC_CHUNK = 1056
SC_GROUPS = SC_CHUNK // 16


def _beta_sc_body(boxes_hbm, field_hbm, beta_hbm, field_v, boxes_v, beta_v):
    cid = lax.axis_index("c")
    sid = lax.axis_index("s")
    wid = sid * 2 + cid
    batch = wid // 8
    chunk = wid % 8
    base = jnp.where(chunk == 7, A - SC_CHUNK, chunk * 1048)

    pltpu.sync_copy(field_hbm.at[pl.ds(batch * (HF * WF), HF * WF)], field_v)
    pltpu.sync_copy(
        boxes_hbm.at[pl.ds((batch * A + base) * 4, SC_CHUNK * 4)], boxes_v)

    iota16 = jnp.arange(16, dtype=jnp.int32)
    t_vals = [float(np.float32(p + 0.5) / np.float32(7.0)) for p in range(POOL)]

    def group(g, _):
        a0 = g * 16
        aidx = (a0 + iota16) * 4
        bx1 = plsc.load_gather(boxes_v, [aidx])
        by1 = plsc.load_gather(boxes_v, [aidx + 1])
        bx2 = plsc.load_gather(boxes_v, [aidx + 2])
        by2 = plsc.load_gather(boxes_v, [aidx + 3])
        dx = bx2 - bx1
        dy = by2 - by1

        x0l, x1l, wxl = [], [], []
        yb0l, yb1l, wyl = [], [], []
        for p in range(POOL):
            xs = bx1 + dx * t_vals[p]
            gx = xs / 640.0 * 159.0
            x0i = gx.astype(jnp.int32)
            wx = gx - x0i.astype(jnp.float32)
            x0l.append(jnp.minimum(x0i, WF - 1))
            x1l.append(jnp.minimum(x0i + 1, WF - 1))
            wxl.append(wx)
            ys = by1 + dy * t_vals[p]
            gy = ys / 640.0 * 159.0
            y0i = gy.astype(jnp.int32)
            wy = gy - y0i.astype(jnp.float32)
            yb0l.append(jnp.minimum(y0i, HF - 1) * WF)
            yb1l.append(jnp.minimum(y0i + 1, HF - 1) * WF)
            wyl.append(wy)

        acc = jnp.zeros((16,), jnp.float32)
        for py in range(POOL):
            wy = wyl[py]
            for px in range(POOL):
                wx = wxl[px]
                v00 = plsc.load_gather(field_v, [yb0l[py] + x0l[px]])
                v01 = plsc.load_gather(field_v, [yb0l[py] + x1l[px]])
                v10 = plsc.load_gather(field_v, [yb1l[py] + x0l[px]])
                v11 = plsc.load_gather(field_v, [yb1l[py] + x1l[px]])
                val = (v00 * (1 - wx) * (1 - wy) + v01 * wx * (1 - wy)
                       + v10 * (1 - wx) * wy + v11 * wx * wy)
                acc = acc + val
        beta_v[pl.ds(a0, 16)] = acc / 49.0
        return 0

    lax.fori_loop(0, SC_GROUPS, group, 0)
    pltpu.sync_copy(beta_v, beta_hbm.at[pl.ds(batch * A + base, SC_CHUNK)])


@functools.partial(
    pl.kernel,
    out_type=jax.ShapeDtypeStruct((B * A,), jnp.float32),
    mesh=plsc.VectorSubcoreMesh(core_axis_name="c", subcore_axis_name="s"),
    compiler_params=pltpu.CompilerParams(needs_layout_passes=False),
    scratch_types=[
        pltpu.VMEM((HF * WF,), jnp.float32),
        pltpu.VMEM((SC_CHUNK * 4,), jnp.float32),
        pltpu.VMEM((SC_CHUNK,), jnp.float32),
    ],
)
def _beta_sc_flat(boxes_hbm, field_hbm, beta_hbm, field_v, boxes_v, beta_v):
    _beta_sc_body(boxes_hbm, field_hbm, beta_hbm, field_v, boxes_v, beta_v)


def _beta_sc(boxes_xyxy, field):
    flat = _beta_sc_flat(jnp.reshape(boxes_xyxy, (B * A * 4,)),
                         jnp.reshape(field, (B * HF * WF,)))
    return jnp.reshape(flat, (B, A))


def _beta_jax(boxes_xyxy, field):
    t = (jnp.arange(POOL, dtype=jnp.float32) + 0.5) / POOL

    def one(f, boxes):
        x1, y1, x2, y2 = boxes[:, 0], boxes[:, 1], boxes[:, 2], boxes[:, 3]
        xs = x1[:, None] + (x2 - x1)[:, None] * t[None, :]
        ys = y1[:, None] + (y2 - y1)[:, None] * t[None, :]
        gx = xs / W * (WF - 1)
        gy = ys / H * (HF - 1)
        gxg = jnp.broadcast_to(gx[:, None, :], (gx.shape[0], POOL, POOL))
        gyg = jnp.broadcast_to(gy[:, :, None], (gy.shape[0], POOL, POOL))
        x0f = jnp.floor(gxg)
        y0f = jnp.floor(gyg)
        x0 = jnp.clip(x0f, 0, WF - 1).astype(jnp.int32)
        x1i = jnp.clip(x0f + 1, 0, WF - 1).astype(jnp.int32)
        y0 = jnp.clip(y0f, 0, HF - 1).astype(jnp.int32)
        y1i = jnp.clip(y0f + 1, 0, HF - 1).astype(jnp.int32)
        wx = gxg - x0f
        wy = gyg - y0f
        v00 = f[y0, x0]
        v01 = f[y0, x1i]
        v10 = f[y1i, x0]
        v11 = f[y1i, x1i]
        vals = (v00 * (1 - wx) * (1 - wy) + v01 * wx * (1 - wy)
                + v10 * (1 - wx) * wy + v11 * wx * wy)
        return vals.mean(axis=(1, 2))

    return jax.vmap(one)(field[:, 0], boxes_xyxy)


def _tc_body(boxes_ref, obj_ref, cls_ref, beta_ref,
             boxes_out, scores_out, classes_out, beta_out, attr_ref):
    boxes = boxes_ref[0]
    obj = obj_ref[0, 0]
    cls_l = cls_ref[0]
    beta = beta_ref[0, 0]

    cal_obj = obj - LAMBDA_OBJ * beta
    sig_obj = jax.nn.sigmoid(cal_obj)
    cls_score = jax.nn.sigmoid(cls_l - LAMBDA_CLS * beta[:, None])
    best_sig = jnp.max(cls_score, axis=-1)
    best_cls = jnp.argmax(cls_score, axis=-1)

    x1 = boxes[:, 0]
    y1 = boxes[:, 1]
    x2 = boxes[:, 2]
    y2 = boxes[:, 3]
    areas_frac = (x2 - x1) * (y2 - y1) / float(H * W)
    small = (areas_frac < A_MIN).astype(jnp.float32)
    beta_small = beta * (1.0 + LAMBDA_SMALL * small)

    combined = sig_obj * best_sig
    masked = jnp.where(combined >= SCORE_THR, combined, NEG_FILL)

    masked_p = jnp.concatenate(
        [masked, jnp.full((AP - A,), PAD_FILL, jnp.float32)])
    v2d = jnp.reshape(masked_p, (ROWS, 128))

    attr_ref[...] = jnp.zeros((AP, 8), jnp.float32)
    attr_ref[0:A, 0] = x1
    attr_ref[0:A, 1] = y1
    attr_ref[0:A, 2] = x2
    attr_ref[0:A, 3] = y2
    attr_ref[0:A, 4] = best_cls.astype(jnp.float32)
    attr_ref[0:A, 5] = beta_small

    idx2d = (lax.broadcasted_iota(jnp.int32, (ROWS, 128), 0) * 128
             + lax.broadcasted_iota(jnp.int32, (ROWS, 128), 1))
    lane = lax.broadcasted_iota(jnp.int32, (1, KP), 1)
    subl = lax.broadcasted_iota(jnp.int32, (KP, 1), 0)

    def body(j, carry):
        v, vals, idx_col = carry
        m = jnp.max(v)
        mi = jnp.min(jnp.where(v == m, idx2d, jnp.int32(2**30)))
        vals = jnp.where(lane == j, m, vals)
        idx_col = jnp.where(subl == j, mi, idx_col)
        v = jnp.where(idx2d == mi, PAD_FILL, v)
        return v, vals, idx_col

    vals0 = jnp.full((1, KP), PAD_FILL, jnp.float32)
    idx0 = jnp.zeros((KP, 1), jnp.int32)
    _, vals, idx_col = lax.fori_loop(0, TOP_K, body, (v2d, vals0, idx0))

    citer = lax.broadcasted_iota(jnp.int32, (KP, CHUNK), 1)
    gathered = jnp.zeros((KP, 8), jnp.float32)
    for c in range(AP // CHUNK):
        oh = (idx_col == citer + c * CHUNK).astype(jnp.float32)
        blk = attr_ref[c * CHUNK:(c + 1) * CHUNK, :]
        gathered = gathered + lax.dot_general(
            oh, blk, (((1,), (0,)), ((), ())),
            precision=lax.Precision.HIGHEST,
            preferred_element_type=jnp.float32)

    valid = vals > -1e8
    scores = jnp.where(valid, vals, 0.0)
    boxes_out[0] = gathered[:TOP_K, 0:4]
    scores_out[0, 0] = scores[0, :TOP_K]
    classes_out[0, 0] = gathered[:TOP_K, 4].astype(jnp.int32)
    beta_col = jnp.reshape(gathered[:, 5], (1, KP))
    beta_out[0, 0] = jnp.where(valid, beta_col, 0.0)[0, :TOP_K]


def _score_topk(boxes_xyxy, obj_logits, cls_logits, beta):
    return pl.pallas_call(
        _tc_body,
        grid=(B,),
        in_specs=[
            pl.BlockSpec((1, A, 4), lambda b: (b, 0, 0)),
            pl.BlockSpec((1, 1, A), lambda b: (b, 0, 0)),
            pl.BlockSpec((1, A, C), lambda b: (b, 0, 0)),
            pl.BlockSpec((1, 1, A), lambda b: (b, 0, 0)),
        ],
        out_specs=[
            pl.BlockSpec((1, TOP_K, 4), lambda b: (b, 0, 0)),
            pl.BlockSpec((1, 1, TOP_K), lambda b: (b, 0, 0)),
            pl.BlockSpec((1, 1, TOP_K), lambda b: (b, 0, 0)),
            pl.BlockSpec((1, 1, TOP_K), lambda b: (b, 0, 0)),
        ],
        out_shape=[
            jax.ShapeDtypeStruct((B, TOP_K, 4), jnp.float32),
            jax.ShapeDtypeStruct((B, 1, TOP_K), jnp.float32),
            jax.ShapeDtypeStruct((B, 1, TOP_K), jnp.int32),
            jax.ShapeDtypeStruct((B, 1, TOP_K), jnp.float32),
        ],
        scratch_shapes=[pltpu.VMEM((AP, 8), jnp.float32)],
    )(boxes_xyxy, obj_logits[:, None, :], cls_logits, beta[:, None, :])


@jax.jit
def kernel(boxes_xyxy, obj_logits, cls_logits, field):
    beta = _beta_sc(boxes_xyxy, field)
    boxes_sel, scores, classes_sel, beta_sel = _score_topk(
        boxes_xyxy, obj_logits, cls_logits, beta)
    return (boxes_sel, scores[:, 0, :], classes_sel[:, 0, :],
            beta_sel[:, 0, :])
